# baseline (speedup 1.0000x reference)
import jax
import jax.numpy as jnp
from jax import lax
from jax.experimental import pallas as pl
from jax.experimental.pallas import tpu as pltpu
from jax.experimental.pallas import tpu_sc as plsc

N_ = 10000
E_ = 320000
ES_ = 100000
DD = 128
NC, NS = 2, 16
NW = NC * NS
NPW = 320
NPAD = NW * NPW
RPLEN = 10264
SUP = 504
SIB = 520
NEGF = -3.0e38
RSQ = 0.08838834764831845
BPW = 3136
ESP = NW * BPW
CH = 56
NCH = BPW // CH
RB = 512
G1 = ESP // RB

_GDN = lax.GatherDimensionNumbers(
    offset_dims=(), collapsed_slice_dims=(0,), start_index_map=(0,))

_BREV = [0, 8, 4, 12, 2, 10, 6, 14, 1, 9, 5, 13, 3, 11, 7, 15]


def _bcast(vec, l):
    idx = jnp.full((16, 1), l, jnp.int32)
    return lax.gather(vec, idx, _GDN, (1,),
                      mode=lax.GatherScatterMode.PROMISE_IN_BOUNDS)


def _perm(v, d, sub, iota16):
    pidx = ((iota16 & (-sub)) + ((iota16 + d) & (sub - 1))).reshape(16, 1)
    return lax.gather(v, pidx, _GDN, (1,),
                      mode=lax.GatherScatterMode.PROMISE_IN_BOUNDS)


def _tree_max(v, iota16):
    for d in (8, 4, 2, 1):
        v = jnp.maximum(v, _perm(v, d, 16, iota16))
    return v


def _tree_sum(v, iota16):
    for d in (8, 4, 2, 1):
        v = v + _perm(v, d, 16, iota16)
    return v


def _att_reduce(ps, iota16):
    cur = ps
    for d, sub in ((8, 16), (4, 8), (2, 4), (1, 2)):
        msk = (iota16 % sub) < d
        nxt = []
        for i in range(len(cur) // 2):
            xr = cur[2 * i] + _perm(cur[2 * i], d, sub, iota16)
            yr = cur[2 * i + 1] + _perm(cur[2 * i + 1], d, sub, iota16)
            nxt.append(jnp.where(msk, xr, _perm(yr, d, sub, iota16)))
        cur = nxt
    return cur[0]


def _edge_sc_body(k_hbm, q_hbm, v_hbm, si_hbm, rp_hbm, agg_hbm,
                  qloc, sibufA, sibufB, attbuf, rploc, aggbuf,
                  kA, kB, vA, vB,
                  semKA, semKB, semVA, semVB, semSA, semSB):
    cid = lax.axis_index("c")
    sid = lax.axis_index("s")
    wid = sid * NC + cid
    base = pl.multiple_of(wid * NPW, 8)
    fbase = pl.multiple_of(wid * (NPW * DD), 8)
    pltpu.sync_copy(q_hbm.at[pl.ds(fbase, NPW * DD)], qloc)
    pltpu.sync_copy(rp_hbm.at[pl.ds(base, 344)], rploc)
    z16 = jnp.zeros((16,), jnp.float32)
    iota16 = lax.iota(jnp.int32, 16)

    def _zb(i, c):
        aggbuf[pl.ds(i * 16, 16)] = z16
        return c
    lax.fori_loop(0, NPW * 8, _zb, 0)

    def _rp(j):
        return rploc[pl.ds(j, 16)][0]

    def _wait_row(buf, sem):
        pltpu.make_async_copy(k_hbm.at[pl.ds(0, 32)], buf, sem).wait()

    def _issue_si(d, sibuf, sem):
        a = _rp(d)
        a8 = (a // 8) * 8
        pltpu.async_copy(si_hbm.at[pl.ds(a8, SIB)], sibuf, sem)

    def _wait_si(sibuf, sem):
        pltpu.make_async_copy(si_hbm.at[pl.ds(0, SIB)], sibuf, sem).wait()

    def _process(d, sibuf):
        a = _rp(d)
        b = _rp(d + 1)
        cnt = b - a
        dq = d * DD

        @pl.when(cnt > 0)
        def _():
            a8 = pl.multiple_of((a // 8) * 8, 8)
            nsup = (b - a8 + (SUP - 1)) // SUP
            qv = [qloc[pl.ds(dq + c8 * 16, 16)] for c8 in range(8)]

            def sup_body(s_i, carry):
                m0 = carry[0]
                s16 = carry[1]
                vc = list(carry[2:])
                st = pl.multiple_of(a8 + s_i * SUP, 8)
                blen = jnp.minimum(jnp.int32(SUP), b - st)

                @pl.when(s_i > 0)
                def _():
                    pltpu.sync_copy(si_hbm.at[pl.ds(st, SIB)], sibuf)

                ng32 = (blen + 31) // 32
                npair32 = (ng32 + 1) // 2

                def _issue(tab, g, buf, sem):
                    idx = sibuf.at[pl.ds(pl.multiple_of(g * 32, 32), 32)]
                    pltpu.async_copy(tab.at[idx], buf, sem)

                _issue(k_hbm, 0, kA, semKA)
                _issue(v_hbm, 0, vA, semVA)

                @pl.when(ng32 > 1)
                def _():
                    _issue(k_hbm, 1, kB, semKB)
                    _issue(v_hbm, 1, vB, semVB)

                def _att16(g16, buf, rb, mx):
                    ps = []
                    for l in _BREV:
                        p = buf[rb + l, pl.ds(0, 16)] * qv[0]
                        for c8 in range(1, 8):
                            p = p + buf[rb + l, pl.ds(c8 * 16, 16)] * qv[c8]
                        ps.append(p)
                    av = _att_reduce(ps, iota16)
                    eid = st + g16 * 16 + iota16
                    valid = (eid >= a) & (eid < b)
                    att = jnp.where(valid, av * RSQ, NEGF)
                    attbuf[pl.ds(g16 * 16, 16)] = att
                    return jnp.maximum(mx, att)

                def _att32(g, buf, mx):
                    mx = _att16(2 * g, buf, 0, mx)
                    return _att16(2 * g + 1, buf, 16, mx)

                def p1_body(p, mx):
                    g0 = 2 * p
                    g1 = g0 + 1
                    _wait_row(kA, semKA)
                    mx = _att32(g0, kA, mx)

                    @pl.when(g0 + 2 < ng32)
                    def _():
                        _issue(k_hbm, g0 + 2, kA, semKA)

                    @pl.when(g1 < ng32)
                    def _():
                        _wait_row(kB, semKB)
                    mx = _att32(g1, kB, mx)

                    @pl.when(g1 + 2 < ng32)
                    def _():
                        _issue(k_hbm, g1 + 2, kB, semKB)
                    return mx

                mx16 = lax.fori_loop(0, npair32, p1_body,
                                     jnp.full((16,), NEGF, jnp.float32))

                msup = _tree_max(mx16, iota16)
                m1 = jnp.maximum(m0, msup)
                sc16 = jnp.exp(m0 - m1)
                s16 = s16 * sc16
                vc = [u * sc16 for u in vc]

                def _acc16(g16, buf, rb, s16c, vcc):
                    att = attbuf[pl.ds(g16 * 16, 16)]
                    e = jnp.exp(att - m1)
                    s16c = s16c + e
                    for l in range(16):
                        w = _bcast(e, l)
                        vcc = [vcc[c8] + w * buf[rb + l, pl.ds(c8 * 16, 16)]
                               for c8 in range(8)]
                    return s16c, vcc

                def _acc32(g, buf, s16c, vcc):
                    s16c, vcc = _acc16(2 * g, buf, 0, s16c, vcc)
                    return _acc16(2 * g + 1, buf, 16, s16c, vcc)

                def p2_body(p, carry2):
                    s16c = carry2[0]
                    vcc = list(carry2[1:])
                    g0 = 2 * p
                    g1 = g0 + 1
                    _wait_row(vA, semVA)
                    s16c, vcc = _acc32(g0, vA, s16c, vcc)

                    @pl.when(g0 + 2 < ng32)
                    def _():
                        _issue(v_hbm, g0 + 2, vA, semVA)

                    @pl.when(g1 < ng32)
                    def _():
                        _wait_row(vB, semVB)
                    s16c, vcc = _acc32(g1, vB, s16c, vcc)

                    @pl.when(g1 + 2 < ng32)
                    def _():
                        _issue(v_hbm, g1 + 2, vB, semVB)
                    return tuple([s16c] + vcc)

                res = lax.fori_loop(0, npair32, p2_body, tuple([s16] + vc))
                return tuple([m1, res[0]] + list(res[1:]))

            init = tuple([jnp.full((16,), NEGF, jnp.float32), z16] + [z16] * 8)
            fin = lax.fori_loop(0, nsup, sup_body, init)
            stot = _tree_sum(fin[1], iota16)
            inv16 = 1.0 / (stot + 1e-9)
            for c8 in range(8):
                aggbuf[pl.ds(dq + c8 * 16, 16)] = fin[2 + c8] * inv16

    _issue_si(0, sibufA, semSA)
    _issue_si(1, sibufB, semSB)

    def dpair(t, c):
        d0 = 2 * t
        _wait_si(sibufA, semSA)
        _process(d0, sibufA)

        @pl.when(t + 1 < NPW // 2)
        def _():
            _issue_si(d0 + 2, sibufA, semSA)
        _wait_si(sibufB, semSB)
        _process(d0 + 1, sibufB)

        @pl.when(t + 1 < NPW // 2)
        def _():
            _issue_si(d0 + 3, sibufB, semSB)
        return c

    lax.fori_loop(0, NPW // 2, dpair, 0)
    pltpu.sync_copy(aggbuf, agg_hbm.at[pl.ds(fbase, NPW * DD)])


_EDGE_CALL = None


def _get_edge_call():
    global _EDGE_CALL
    if _EDGE_CALL is None:
        _EDGE_CALL = pl.kernel(
            _edge_sc_body,
            out_type=jax.ShapeDtypeStruct((NPAD * DD,), jnp.float32),
            mesh=plsc.VectorSubcoreMesh(core_axis_name="c",
                                        subcore_axis_name="s",
                                        num_cores=NC, num_subcores=NS),
            scratch_types=[
                pltpu.VMEM((NPW * DD,), jnp.float32),
                pltpu.VMEM((SIB,), jnp.int32),
                pltpu.VMEM((SIB,), jnp.int32),
                pltpu.VMEM((512,), jnp.float32),
                pltpu.VMEM((344,), jnp.int32),
                pltpu.VMEM((NPW * DD,), jnp.float32),
                pltpu.VMEM((32, DD), jnp.float32),
                pltpu.VMEM((32, DD), jnp.float32),
                pltpu.VMEM((32, DD), jnp.float32),
                pltpu.VMEM((32, DD), jnp.float32),
                pltpu.SemaphoreType.DMA,
                pltpu.SemaphoreType.DMA,
                pltpu.SemaphoreType.DMA,
                pltpu.SemaphoreType.DMA,
                pltpu.SemaphoreType.DMA,
                pltpu.SemaphoreType.DMA,
            ],
        )
    return _EDGE_CALL


def _edge(k, q, v, sip, rp):
    qp = jnp.pad(q, ((0, NPAD - N_), (0, 0))).reshape(NPAD * DD)
    agg = _get_edge_call()(k, qp, v, sip, rp)
    return agg.reshape(NPAD, DD)[:N_]


def _gath_body(df, gf, isrc, idst, hd, hg,
               ibD, ibG, bD0, bD1, bG0, bG1, sD0, sD1, sG0, sG1):
    cid = lax.axis_index("c")
    sid = lax.axis_index("s")
    wid = sid * NC + cid
    rbase = pl.multiple_of(wid * BPW, 8)
    pltpu.sync_copy(isrc.at[pl.ds(rbase, BPW)], ibD)
    pltpu.sync_copy(idst.at[pl.ds(rbase, BPW)], ibG)

    def _issue(c, bD, sD, bG, sG):
        pltpu.async_copy(df.at[ibD.at[pl.ds(c * CH, CH)]], bD, sD)
        pltpu.async_copy(gf.at[ibG.at[pl.ds(c * CH, CH)]], bG, sG)

    def _wait(buf, sem):
        pltpu.make_async_copy(df.at[pl.ds(0, CH)], buf, sem).wait()

    def _out(c, bD, bG):
        pltpu.sync_copy(bD, hd.at[pl.ds(rbase + c * CH, CH)])
        pltpu.sync_copy(bG, hg.at[pl.ds(rbase + c * CH, CH)])

    _issue(0, bD0, sD0, bG0, sG0)
    _issue(1, bD1, sD1, bG1, sG1)

    def pair(t, c):
        c0 = 2 * t
        _wait(bD0, sD0)
        _wait(bG0, sG0)
        _out(c0, bD0, bG0)

        @pl.when(t + 1 < NCH // 2)
        def _():
            _issue(c0 + 2, bD0, sD0, bG0, sG0)
        _wait(bD1, sD1)
        _wait(bG1, sG1)
        _out(c0 + 1, bD1, bG1)

        @pl.when(t + 1 < NCH // 2)
        def _():
            _issue(c0 + 3, bD1, sD1, bG1, sG1)
        return c

    lax.fori_loop(0, NCH // 2, pair, 0)


_GATH_CALL = None


def _get_gath_call():
    global _GATH_CALL
    if _GATH_CALL is None:
        _GATH_CALL = pl.kernel(
            _gath_body,
            out_type=[jax.ShapeDtypeStruct((ESP, 512), jnp.float32),
                      jax.ShapeDtypeStruct((ESP, 512), jnp.float32)],
            mesh=plsc.VectorSubcoreMesh(core_axis_name="c",
                                        subcore_axis_name="s",
                                        num_cores=NC, num_subcores=NS),
            scratch_types=[
                pltpu.VMEM((BPW,), jnp.int32),
                pltpu.VMEM((BPW,), jnp.int32),
                pltpu.VMEM((CH, 512), jnp.float32),
                pltpu.VMEM((CH, 512), jnp.float32),
                pltpu.VMEM((CH, 512), jnp.float32),
                pltpu.VMEM((CH, 512), jnp.float32),
                pltpu.SemaphoreType.DMA,
                pltpu.SemaphoreType.DMA,
                pltpu.SemaphoreType.DMA,
                pltpu.SemaphoreType.DMA,
            ],
        )
    return _GATH_CALL


def _proj_body(ph_ref, dh_ref, gh_ref, wp_ref, wd_ref, wg_ref,
               op_ref, od_ref, og_ref):
    op_ref[...] = jnp.dot(ph_ref[...], wp_ref[...],
                          preferred_element_type=jnp.float32)
    od_ref[...] = jnp.dot(dh_ref[...], wd_ref[...],
                          preferred_element_type=jnp.float32)
    og_ref[...] = jnp.dot(gh_ref[...], wg_ref[...],
                          preferred_element_type=jnp.float32)


def _proj(ph, dh, gh, Wp, Wd, Wg):
    B = 400
    G = N_ // B
    return pl.pallas_call(
        _proj_body,
        grid=(G,),
        in_specs=[pl.BlockSpec((B, 128), lambda i: (i, 0)),
                  pl.BlockSpec((B, 128), lambda i: (i, 0)),
                  pl.BlockSpec((B, 128), lambda i: (i, 0)),
                  pl.BlockSpec((128, 768), lambda i: (0, 0)),
                  pl.BlockSpec((128, 384), lambda i: (0, 0)),
                  pl.BlockSpec((128, 384), lambda i: (0, 0))],
        out_specs=[pl.BlockSpec((B, 768), lambda i: (i, 0)),
                   pl.BlockSpec((B, 384), lambda i: (i, 0)),
                   pl.BlockSpec((B, 384), lambda i: (i, 0))],
        out_shape=[jax.ShapeDtypeStruct((N_, 768), jnp.float32),
                   jax.ShapeDtypeStruct((N_, 384), jnp.float32),
                   jax.ShapeDtypeStruct((N_, 384), jnp.float32)],
    )(ph, dh, gh, Wp, Wd, Wg)


def _post_body(a0_ref, a1_ref, a2_ref, a3_ref, ph_ref, dh_ref, gh_ref,
               wa_ref, lg_ref, lb_ref, np_ref, nd_ref, ng_ref):
    def hgt_out(agg, h, j):
        o = jnp.dot(jax.nn.gelu(agg), wa_ref[j],
                    preferred_element_type=jnp.float32)
        o = 0.5 * o + 0.5 * h
        mu = jnp.mean(o, axis=-1, keepdims=True)
        var = jnp.mean((o - mu) ** 2, axis=-1, keepdims=True)
        return lg_ref[j] * (o - mu) / jnp.sqrt(var + 1e-5) + lb_ref[j]

    tp_d = hgt_out(a0_ref[...], ph_ref[...], 0)
    tp_g = hgt_out(a1_ref[...], ph_ref[...], 1)
    nd = hgt_out(a2_ref[...], dh_ref[...], 2)
    ng2 = hgt_out(a3_ref[...], gh_ref[...], 3)
    t0 = jnp.max(tp_d, axis=-1, keepdims=True)
    t1 = jnp.max(tp_g, axis=-1, keepdims=True)
    mm = jnp.maximum(t0, t1)
    e0 = jnp.exp(t0 - mm)
    e1 = jnp.exp(t1 - mm)
    np_ref[...] = (e0 * tp_d + e1 * tp_g) / (e0 + e1)
    nd_ref[...] = nd
    ng_ref[...] = ng2


def _post(a0, a1, a2, a3, ph, dh, gh, wa, lg, lb):
    B = 400
    G = N_ // B
    bs = pl.BlockSpec((B, 128), lambda i: (i, 0))
    return pl.pallas_call(
        _post_body,
        grid=(G,),
        in_specs=[bs, bs, bs, bs, bs, bs, bs,
                  pl.BlockSpec((4, 128, 128), lambda i: (0, 0, 0)),
                  pl.BlockSpec((4, 1, 128), lambda i: (0, 0, 0)),
                  pl.BlockSpec((4, 1, 128), lambda i: (0, 0, 0))],
        out_specs=[bs, bs, bs],
        out_shape=[jax.ShapeDtypeStruct((N_, 128), jnp.float32)] * 3,
    )(a0, a1, a2, a3, ph, dh, gh, wa, lg, lb)


def _mlp1_body(hd_ref, hg_ref, w1a_ref, w1b_ref, b1_ref, y_ref, s_ref,
               acc_ref):
    i = pl.program_id(0)

    @pl.when(i == 0)
    def _():
        acc_ref[...] = jnp.zeros_like(acc_ref)

    y = jnp.dot(hd_ref[...], w1a_ref[...], preferred_element_type=jnp.float32)
    y = y + jnp.dot(hg_ref[...], w1b_ref[...],
                    preferred_element_type=jnp.float32) + b1_ref[...]
    y_ref[...] = y
    rid = i * RB + lax.broadcasted_iota(jnp.int32, (RB, 1), 0)
    msk = rid < ES_
    ym = jnp.where(msk, y, 0.0)
    y2 = jnp.where(msk, y * y, 0.0)
    acc_ref[...] += jnp.concatenate(
        [jnp.sum(ym, axis=0, keepdims=True),
         jnp.sum(y2, axis=0, keepdims=True)], axis=0)

    @pl.when(i == G1 - 1)
    def _():
        s_ref[...] = acc_ref[...]


def _mlp1(hd, hg, w1a, w1b, b1):
    return pl.pallas_call(
        _mlp1_body,
        grid=(G1,),
        in_specs=[pl.BlockSpec((RB, 512), lambda i: (i, 0)),
                  pl.BlockSpec((RB, 512), lambda i: (i, 0)),
                  pl.BlockSpec((512, 512), lambda i: (0, 0)),
                  pl.BlockSpec((512, 512), lambda i: (0, 0)),
                  pl.BlockSpec((1, 512), lambda i: (0, 0))],
        out_specs=[pl.BlockSpec((RB, 512), lambda i: (i, 0)),
                   pl.BlockSpec((2, 512), lambda i: (0, 0))],
        out_shape=[jax.ShapeDtypeStruct((ESP, 512), jnp.float32),
                   jax.ShapeDtypeStruct((2, 512), jnp.float32)],
        scratch_shapes=[pltpu.VMEM((2, 512), jnp.float32)],
    )(hd, hg, w1a, w1b, b1)


def _mlp2_body(y1_ref, sc_ref, sh_ref, w2_ref, b2_ref, y_ref, s_ref,
               acc_ref):
    i = pl.program_id(0)

    @pl.when(i == 0)
    def _():
        acc_ref[...] = jnp.zeros_like(acc_ref)

    x1 = jnp.maximum(y1_ref[...] * sc_ref[...] + sh_ref[...], 0.0)
    y = jnp.dot(x1, w2_ref[...], preferred_element_type=jnp.float32) \
        + b2_ref[...]
    y_ref[...] = y
    rid = i * RB + lax.broadcasted_iota(jnp.int32, (RB, 1), 0)
    msk = rid < ES_
    ym = jnp.where(msk, y, 0.0)
    y2 = jnp.where(msk, y * y, 0.0)
    acc_ref[...] += jnp.concatenate(
        [jnp.sum(ym, axis=0, keepdims=True),
         jnp.sum(y2, axis=0, keepdims=True)], axis=0)

    @pl.when(i == G1 - 1)
    def _():
        s_ref[...] = acc_ref[...]


def _mlp2(y1, sc1, sh1, w2, b2):
    return pl.pallas_call(
        _mlp2_body,
        grid=(G1,),
        in_specs=[pl.BlockSpec((RB, 512), lambda i: (i, 0)),
                  pl.BlockSpec((1, 512), lambda i: (0, 0)),
                  pl.BlockSpec((1, 512), lambda i: (0, 0)),
                  pl.BlockSpec((512, 256), lambda i: (0, 0)),
                  pl.BlockSpec((1, 256), lambda i: (0, 0))],
        out_specs=[pl.BlockSpec((RB, 256), lambda i: (i, 0)),
                   pl.BlockSpec((2, 256), lambda i: (0, 0))],
        out_shape=[jax.ShapeDtypeStruct((ESP, 256), jnp.float32),
                   jax.ShapeDtypeStruct((2, 256), jnp.float32)],
        scratch_shapes=[pltpu.VMEM((2, 256), jnp.float32)],
    )(y1, sc1, sh1, w2, b2)


def _mlp3_body(y2_ref, sc_ref, sh_ref, w3_ref, b3_ref, o_ref):
    x2 = jnp.maximum(y2_ref[...] * sc_ref[...] + sh_ref[...], 0.0)
    z = jnp.sum(x2 * w3_ref[...], axis=-1, keepdims=True) + b3_ref[...]
    o_ref[...] = jax.nn.sigmoid(z)


def _mlp3(y2, sc2, sh2, w3, b3):
    return pl.pallas_call(
        _mlp3_body,
        grid=(G1,),
        in_specs=[pl.BlockSpec((RB, 256), lambda i: (i, 0)),
                  pl.BlockSpec((1, 256), lambda i: (0, 0)),
                  pl.BlockSpec((1, 256), lambda i: (0, 0)),
                  pl.BlockSpec((1, 256), lambda i: (0, 0)),
                  pl.BlockSpec((1, 1), lambda i: (0, 0))],
        out_specs=pl.BlockSpec((RB, 1), lambda i: (i, 0)),
        out_shape=jax.ShapeDtypeStruct((ESP, 1), jnp.float32),
    )(y2, sc2, sh2, w3, b3)


def _prep(si, di):
    rp = jnp.searchsorted(
        di.astype(jnp.int32),
        jnp.arange(RPLEN, dtype=jnp.int32)).astype(jnp.int32)
    sip = jnp.concatenate([si.astype(jnp.int32),
                           jnp.zeros((528,), jnp.int32)])
    return sip, rp


def kernel(prot_h0, drug_h0, dis_h0, dt_src, dt_dst, dg_src, dg_dst,
           td_src, td_dst, gd_src, gd_dst, score_src, score_dst,
           Wk, Wq, Wv, Wa, lng, lnb, W1, bb1, g1, be1, W2, bb2, g2, be2,
           W3, bb3):
    dt = _prep(dt_src, dt_dst)
    dg = _prep(dg_src, dg_dst)
    td = _prep(td_src, td_dst)
    gd = _prep(gd_src, gd_dst)

    ph, dh, gh = prot_h0, drug_h0, dis_h0
    drug_feats, dis_feats = [], []
    for i in range(4):
        Wp = jnp.concatenate([Wq[i, 0], Wq[i, 1], Wk[i, 2], Wv[i, 2],
                              Wk[i, 3], Wv[i, 3]], axis=1)
        Wd = jnp.concatenate([Wk[i, 0], Wv[i, 0], Wq[i, 2]], axis=1)
        Wg = jnp.concatenate([Wk[i, 1], Wv[i, 1], Wq[i, 3]], axis=1)
        pj_p, pj_d, pj_g = _proj(ph, dh, gh, Wp, Wd, Wg)
        q_dt, q_dg = pj_p[:, 0:128], pj_p[:, 128:256]
        k_td, v_td = pj_p[:, 256:384], pj_p[:, 384:512]
        k_gd, v_gd = pj_p[:, 512:640], pj_p[:, 640:768]
        k_dt, v_dt, q_td = pj_d[:, 0:128], pj_d[:, 128:256], pj_d[:, 256:384]
        k_dg, v_dg, q_gd = pj_g[:, 0:128], pj_g[:, 128:256], pj_g[:, 256:384]
        agg0 = _edge(k_dt, q_dt, v_dt, *dt)
        agg1 = _edge(k_dg, q_dg, v_dg, *dg)
        agg2 = _edge(k_td, q_td, v_td, *td)
        agg3 = _edge(k_gd, q_gd, v_gd, *gd)
        ph, dh, gh = _post(agg0, agg1, agg2, agg3, ph, dh, gh,
                           Wa[i], lng[i][:, None, :], lnb[i][:, None, :])
        drug_feats.append(dh)
        dis_feats.append(gh)

    df = jnp.concatenate(drug_feats, axis=1)
    gf = jnp.concatenate(dis_feats, axis=1)
    ssrc = jnp.concatenate([score_src.astype(jnp.int32),
                            jnp.zeros((ESP - ES_,), jnp.int32)])
    sdst = jnp.concatenate([score_dst.astype(jnp.int32),
                            jnp.zeros((ESP - ES_,), jnp.int32)])
    hd, hg = _get_gath_call()(df, gf, ssrc, sdst)

    y1, s1 = _mlp1(hd, hg, W1[:512], W1[512:], bb1[None, :])
    mu1 = s1[0] / ES_
    var1 = s1[1] / ES_ - mu1 * mu1
    rs1 = 1.0 / jnp.sqrt(var1 + 1e-5)
    sc1 = (g1 * rs1)[None, :]
    sh1 = (be1 - g1 * rs1 * mu1)[None, :]

    y2, s2 = _mlp2(y1, sc1, sh1, W2, bb2[None, :])
    mu2 = s2[0] / ES_
    var2 = s2[1] / ES_ - mu2 * mu2
    rs2 = 1.0 / jnp.sqrt(var2 + 1e-5)
    sc2 = (g2 * rs2)[None, :]
    sh2 = (be2 - g2 * rs2 * mu2)[None, :]

    out = _mlp3(y2, sc2, sh2, W3[:, 0][None, :], bb3[None, :])
    return out.reshape(ESP)[:ES_]


# speedup vs baseline: 1.6491x; 1.6491x over previous
import jax
import jax.numpy as jnp
from jax import lax
from jax.experimental import pallas as pl
from jax.experimental.pallas import tpu as pltpu
from jax.experimental.pallas import tpu_sc as plsc

N_ = 10000
E_ = 320000
ES_ = 100000
DD = 128
NC, NS = 2, 16
NW = NC * NS
NPW = 320
NPAD = NW * NPW
RPLEN = 10264
SUP = 504
SIB = 520
NEGF = -3.0e38
RSQ = 0.08838834764831845
BPW = 3136
ESP = NW * BPW
CH = 56
NCH = BPW // CH
RB = 512
G1 = ESP // RB

_GDN = lax.GatherDimensionNumbers(
    offset_dims=(), collapsed_slice_dims=(0,), start_index_map=(0,))

_BREV = [0, 8, 4, 12, 2, 10, 6, 14, 1, 9, 5, 13, 3, 11, 7, 15]


def _bcast(vec, l):
    idx = jnp.full((16, 1), l, jnp.int32)
    return lax.gather(vec, idx, _GDN, (1,),
                      mode=lax.GatherScatterMode.PROMISE_IN_BOUNDS)


def _perm(v, d, sub, iota16):
    pidx = ((iota16 & (-sub)) + ((iota16 + d) & (sub - 1))).reshape(16, 1)
    return lax.gather(v, pidx, _GDN, (1,),
                      mode=lax.GatherScatterMode.PROMISE_IN_BOUNDS)


def _tree_max(v, iota16):
    for d in (8, 4, 2, 1):
        v = jnp.maximum(v, _perm(v, d, 16, iota16))
    return v


def _tree_sum(v, iota16):
    for d in (8, 4, 2, 1):
        v = v + _perm(v, d, 16, iota16)
    return v


def _att_reduce(ps, iota16):
    cur = ps
    for d, sub in ((8, 16), (4, 8), (2, 4), (1, 2)):
        msk = (iota16 % sub) < d
        nxt = []
        for i in range(len(cur) // 2):
            xr = cur[2 * i] + _perm(cur[2 * i], d, sub, iota16)
            yr = cur[2 * i + 1] + _perm(cur[2 * i + 1], d, sub, iota16)
            nxt.append(jnp.where(msk, xr, _perm(yr, d, sub, iota16)))
        cur = nxt
    return cur[0]


def _edge_sc_body(kv_hbm, q_hbm, si_hbm, rp_hbm, agg_hbm,
                  qloc, sibufA, sibufB, rploc, aggbuf, bufA, bufB,
                  semKA, semKB, semSA, semSB):
    cid = lax.axis_index("c")
    sid = lax.axis_index("s")
    wid = sid * NC + cid
    base = pl.multiple_of(wid * NPW, 8)
    fbase = pl.multiple_of(wid * (NPW * DD), 8)
    pltpu.sync_copy(q_hbm.at[pl.ds(fbase, NPW * DD)], qloc)
    pltpu.sync_copy(rp_hbm.at[pl.ds(base, 344)], rploc)
    z16 = jnp.zeros((16,), jnp.float32)
    iota16 = lax.iota(jnp.int32, 16)

    def _zb(i, c):
        aggbuf[pl.ds(i * 16, 16)] = z16
        return c
    lax.fori_loop(0, NPW * 8, _zb, 0)

    def _rp(j):
        return rploc[pl.ds(j, 16)][0]

    def _wait_kv(buf, sem):
        pltpu.make_async_copy(kv_hbm.at[pl.ds(0, 16)], buf, sem).wait()

    def _issue_si(d, sibuf, sem):
        a = _rp(d)
        a8 = pl.multiple_of((a // 8) * 8, 8)
        pltpu.async_copy(si_hbm.at[pl.ds(a8, SIB)], sibuf, sem)

    def _wait_si(sibuf, sem):
        pltpu.make_async_copy(si_hbm.at[pl.ds(0, SIB)], sibuf, sem).wait()

    def _process(d, sibuf):
        a = _rp(d)
        b = _rp(d + 1)
        cnt = b - a
        dq = d * DD

        @pl.when(cnt > 0)
        def _():
            a8 = pl.multiple_of((a // 8) * 8, 8)
            nsup = (b - a8 + (SUP - 1)) // SUP
            qv = [qloc[pl.ds(dq + c8 * 16, 16)] for c8 in range(8)]

            def sup_body(s_i, carry):
                st = pl.multiple_of(a8 + s_i * SUP, 8)
                blen = jnp.minimum(jnp.int32(SUP), b - st)

                @pl.when(s_i > 0)
                def _():
                    pltpu.sync_copy(si_hbm.at[pl.ds(st, SIB)], sibuf)

                ng = (blen + 15) // 16
                npair = (ng + 1) // 2
                ng2 = npair * 2

                def _issue(g, buf, sem):
                    sivals = sibuf[pl.ds(g * 16, 16)]
                    pltpu.async_copy(kv_hbm.at[sivals], buf, sem)

                _issue(0, bufA, semKA)
                _issue(1, bufB, semKB)

                def _grp(g, buf, m0, s16, vc):
                    ps = []
                    for l in _BREV:
                        p = buf[l, pl.ds(0, 16)] * qv[0]
                        for c8 in range(1, 8):
                            p = p + buf[l, pl.ds(c8 * 16, 16)] * qv[c8]
                        ps.append(p)
                    av = _att_reduce(ps, iota16)
                    eid = st + g * 16 + iota16
                    valid = (eid >= a) & (eid < b)
                    att = jnp.where(valid, av * RSQ, NEGF)
                    m1 = jnp.maximum(m0, _tree_max(att, iota16))
                    sc = jnp.exp(m0 - m1)
                    e = jnp.exp(att - m1)
                    s16n = s16 * sc + e
                    vc = [u * sc for u in vc]
                    for l in range(16):
                        w = _bcast(e, l)
                        vc = [vc[c8] + w * buf[l, pl.ds(DD + c8 * 16, 16)]
                              for c8 in range(8)]
                    return m1, s16n, vc

                def pair_body(p, carry2):
                    m0 = carry2[0]
                    s16 = carry2[1]
                    vc = list(carry2[2:])
                    g0 = 2 * p
                    _wait_kv(bufA, semKA)
                    m0, s16, vc = _grp(g0, bufA, m0, s16, vc)

                    @pl.when(g0 + 2 < ng2)
                    def _():
                        _issue(g0 + 2, bufA, semKA)
                    _wait_kv(bufB, semKB)
                    m0, s16, vc = _grp(g0 + 1, bufB, m0, s16, vc)

                    @pl.when(g0 + 3 < ng2)
                    def _():
                        _issue(g0 + 3, bufB, semKB)
                    return tuple([m0, s16] + vc)

                return lax.fori_loop(0, npair, pair_body, carry)

            init = tuple([jnp.full((16,), NEGF, jnp.float32), z16] + [z16] * 8)
            fin = lax.fori_loop(0, nsup, sup_body, init)
            stot = _tree_sum(fin[1], iota16)
            inv16 = 1.0 / (stot + 1e-9)
            for c8 in range(8):
                aggbuf[pl.ds(dq + c8 * 16, 16)] = fin[2 + c8] * inv16

    _issue_si(0, sibufA, semSA)
    _issue_si(1, sibufB, semSB)

    def dpair(t, c):
        d0 = 2 * t
        _wait_si(sibufA, semSA)
        _process(d0, sibufA)

        @pl.when(t + 1 < NPW // 2)
        def _():
            _issue_si(d0 + 2, sibufA, semSA)
        _wait_si(sibufB, semSB)
        _process(d0 + 1, sibufB)

        @pl.when(t + 1 < NPW // 2)
        def _():
            _issue_si(d0 + 3, sibufB, semSB)
        return c

    lax.fori_loop(0, NPW // 2, dpair, 0)
    pltpu.sync_copy(aggbuf, agg_hbm.at[pl.ds(fbase, NPW * DD)])


_EDGE_CALL = None


def _get_edge_call():
    global _EDGE_CALL
    if _EDGE_CALL is None:
        _EDGE_CALL = pl.kernel(
            _edge_sc_body,
            out_type=jax.ShapeDtypeStruct((NPAD * DD,), jnp.float32),
            mesh=plsc.VectorSubcoreMesh(core_axis_name="c",
                                        subcore_axis_name="s",
                                        num_cores=NC, num_subcores=NS),
            scratch_types=[
                pltpu.VMEM((NPW * DD,), jnp.float32),
                pltpu.VMEM((SIB,), jnp.int32),
                pltpu.VMEM((SIB,), jnp.int32),
                pltpu.VMEM((344,), jnp.int32),
                pltpu.VMEM((NPW * DD,), jnp.float32),
                pltpu.VMEM((16, 2 * DD), jnp.float32),
                pltpu.VMEM((16, 2 * DD), jnp.float32),
                pltpu.SemaphoreType.DMA,
                pltpu.SemaphoreType.DMA,
                pltpu.SemaphoreType.DMA,
                pltpu.SemaphoreType.DMA,
            ],
        )
    return _EDGE_CALL


def _edge(kv, q, sip, rp):
    qp = jnp.pad(q, ((0, NPAD - N_), (0, 0))).reshape(NPAD * DD)
    agg = _get_edge_call()(kv, qp, sip, rp)
    return agg.reshape(NPAD, DD)[:N_]


def _gath_body(df, gf, isrc, idst, hd, hg,
               ibD, ibG, bD0, bD1, bG0, bG1, sD0, sD1, sG0, sG1):
    cid = lax.axis_index("c")
    sid = lax.axis_index("s")
    wid = sid * NC + cid
    rbase = pl.multiple_of(wid * BPW, 8)
    pltpu.sync_copy(isrc.at[pl.ds(rbase, BPW)], ibD)
    pltpu.sync_copy(idst.at[pl.ds(rbase, BPW)], ibG)

    def _issue(c, bD, sD, bG, sG):
        pltpu.async_copy(df.at[ibD.at[pl.ds(c * CH, CH)]], bD, sD)
        pltpu.async_copy(gf.at[ibG.at[pl.ds(c * CH, CH)]], bG, sG)

    def _wait(buf, sem):
        pltpu.make_async_copy(df.at[pl.ds(0, CH)], buf, sem).wait()

    def _out(c, bD, bG):
        pltpu.sync_copy(bD, hd.at[pl.ds(rbase + c * CH, CH)])
        pltpu.sync_copy(bG, hg.at[pl.ds(rbase + c * CH, CH)])

    _issue(0, bD0, sD0, bG0, sG0)
    _issue(1, bD1, sD1, bG1, sG1)

    def pair(t, c):
        c0 = 2 * t
        _wait(bD0, sD0)
        _wait(bG0, sG0)
        _out(c0, bD0, bG0)

        @pl.when(t + 1 < NCH // 2)
        def _():
            _issue(c0 + 2, bD0, sD0, bG0, sG0)
        _wait(bD1, sD1)
        _wait(bG1, sG1)
        _out(c0 + 1, bD1, bG1)

        @pl.when(t + 1 < NCH // 2)
        def _():
            _issue(c0 + 3, bD1, sD1, bG1, sG1)
        return c

    lax.fori_loop(0, NCH // 2, pair, 0)


_GATH_CALL = None


def _get_gath_call():
    global _GATH_CALL
    if _GATH_CALL is None:
        _GATH_CALL = pl.kernel(
            _gath_body,
            out_type=[jax.ShapeDtypeStruct((ESP, 512), jnp.float32),
                      jax.ShapeDtypeStruct((ESP, 512), jnp.float32)],
            mesh=plsc.VectorSubcoreMesh(core_axis_name="c",
                                        subcore_axis_name="s",
                                        num_cores=NC, num_subcores=NS),
            scratch_types=[
                pltpu.VMEM((BPW,), jnp.int32),
                pltpu.VMEM((BPW,), jnp.int32),
                pltpu.VMEM((CH, 512), jnp.float32),
                pltpu.VMEM((CH, 512), jnp.float32),
                pltpu.VMEM((CH, 512), jnp.float32),
                pltpu.VMEM((CH, 512), jnp.float32),
                pltpu.SemaphoreType.DMA,
                pltpu.SemaphoreType.DMA,
                pltpu.SemaphoreType.DMA,
                pltpu.SemaphoreType.DMA,
            ],
        )
    return _GATH_CALL


def _proj_body(ph_ref, dh_ref, gh_ref, wp_ref, wd_ref, wg_ref,
               op_ref, od_ref, og_ref):
    op_ref[...] = jnp.dot(ph_ref[...], wp_ref[...],
                          preferred_element_type=jnp.float32)
    od_ref[...] = jnp.dot(dh_ref[...], wd_ref[...],
                          preferred_element_type=jnp.float32)
    og_ref[...] = jnp.dot(gh_ref[...], wg_ref[...],
                          preferred_element_type=jnp.float32)


def _proj(ph, dh, gh, Wp, Wd, Wg):
    B = 400
    G = N_ // B
    return pl.pallas_call(
        _proj_body,
        grid=(G,),
        in_specs=[pl.BlockSpec((B, 128), lambda i: (i, 0)),
                  pl.BlockSpec((B, 128), lambda i: (i, 0)),
                  pl.BlockSpec((B, 128), lambda i: (i, 0)),
                  pl.BlockSpec((128, 768), lambda i: (0, 0)),
                  pl.BlockSpec((128, 384), lambda i: (0, 0)),
                  pl.BlockSpec((128, 384), lambda i: (0, 0))],
        out_specs=[pl.BlockSpec((B, 768), lambda i: (i, 0)),
                   pl.BlockSpec((B, 384), lambda i: (i, 0)),
                   pl.BlockSpec((B, 384), lambda i: (i, 0))],
        out_shape=[jax.ShapeDtypeStruct((N_, 768), jnp.float32),
                   jax.ShapeDtypeStruct((N_, 384), jnp.float32),
                   jax.ShapeDtypeStruct((N_, 384), jnp.float32)],
    )(ph, dh, gh, Wp, Wd, Wg)


def _post_body(a0_ref, a1_ref, a2_ref, a3_ref, ph_ref, dh_ref, gh_ref,
               wa_ref, lg_ref, lb_ref, np_ref, nd_ref, ng_ref):
    def hgt_out(agg, h, j):
        o = jnp.dot(jax.nn.gelu(agg), wa_ref[j],
                    preferred_element_type=jnp.float32)
        o = 0.5 * o + 0.5 * h
        mu = jnp.mean(o, axis=-1, keepdims=True)
        var = jnp.mean((o - mu) ** 2, axis=-1, keepdims=True)
        return lg_ref[j] * (o - mu) / jnp.sqrt(var + 1e-5) + lb_ref[j]

    tp_d = hgt_out(a0_ref[...], ph_ref[...], 0)
    tp_g = hgt_out(a1_ref[...], ph_ref[...], 1)
    nd = hgt_out(a2_ref[...], dh_ref[...], 2)
    ng2 = hgt_out(a3_ref[...], gh_ref[...], 3)
    t0 = jnp.max(tp_d, axis=-1, keepdims=True)
    t1 = jnp.max(tp_g, axis=-1, keepdims=True)
    mm = jnp.maximum(t0, t1)
    e0 = jnp.exp(t0 - mm)
    e1 = jnp.exp(t1 - mm)
    np_ref[...] = (e0 * tp_d + e1 * tp_g) / (e0 + e1)
    nd_ref[...] = nd
    ng_ref[...] = ng2


def _post(a0, a1, a2, a3, ph, dh, gh, wa, lg, lb):
    B = 400
    G = N_ // B
    bs = pl.BlockSpec((B, 128), lambda i: (i, 0))
    return pl.pallas_call(
        _post_body,
        grid=(G,),
        in_specs=[bs, bs, bs, bs, bs, bs, bs,
                  pl.BlockSpec((4, 128, 128), lambda i: (0, 0, 0)),
                  pl.BlockSpec((4, 1, 128), lambda i: (0, 0, 0)),
                  pl.BlockSpec((4, 1, 128), lambda i: (0, 0, 0))],
        out_specs=[bs, bs, bs],
        out_shape=[jax.ShapeDtypeStruct((N_, 128), jnp.float32)] * 3,
    )(a0, a1, a2, a3, ph, dh, gh, wa, lg, lb)


def _mlp1_body(hd_ref, hg_ref, w1a_ref, w1b_ref, b1_ref, y_ref, s_ref,
               acc_ref):
    i = pl.program_id(0)

    @pl.when(i == 0)
    def _():
        acc_ref[...] = jnp.zeros_like(acc_ref)

    y = jnp.dot(hd_ref[...], w1a_ref[...], preferred_element_type=jnp.float32)
    y = y + jnp.dot(hg_ref[...], w1b_ref[...],
                    preferred_element_type=jnp.float32) + b1_ref[...]
    y_ref[...] = y
    rid = i * RB + lax.broadcasted_iota(jnp.int32, (RB, 1), 0)
    msk = rid < ES_
    ym = jnp.where(msk, y, 0.0)
    y2 = jnp.where(msk, y * y, 0.0)
    acc_ref[...] += jnp.concatenate(
        [jnp.sum(ym, axis=0, keepdims=True),
         jnp.sum(y2, axis=0, keepdims=True)], axis=0)

    @pl.when(i == G1 - 1)
    def _():
        s_ref[...] = acc_ref[...]


def _mlp1(hd, hg, w1a, w1b, b1):
    return pl.pallas_call(
        _mlp1_body,
        grid=(G1,),
        in_specs=[pl.BlockSpec((RB, 512), lambda i: (i, 0)),
                  pl.BlockSpec((RB, 512), lambda i: (i, 0)),
                  pl.BlockSpec((512, 512), lambda i: (0, 0)),
                  pl.BlockSpec((512, 512), lambda i: (0, 0)),
                  pl.BlockSpec((1, 512), lambda i: (0, 0))],
        out_specs=[pl.BlockSpec((RB, 512), lambda i: (i, 0)),
                   pl.BlockSpec((2, 512), lambda i: (0, 0))],
        out_shape=[jax.ShapeDtypeStruct((ESP, 512), jnp.float32),
                   jax.ShapeDtypeStruct((2, 512), jnp.float32)],
        scratch_shapes=[pltpu.VMEM((2, 512), jnp.float32)],
    )(hd, hg, w1a, w1b, b1)


def _mlp2_body(y1_ref, sc_ref, sh_ref, w2_ref, b2_ref, y_ref, s_ref,
               acc_ref):
    i = pl.program_id(0)

    @pl.when(i == 0)
    def _():
        acc_ref[...] = jnp.zeros_like(acc_ref)

    x1 = jnp.maximum(y1_ref[...] * sc_ref[...] + sh_ref[...], 0.0)
    y = jnp.dot(x1, w2_ref[...], preferred_element_type=jnp.float32) \
        + b2_ref[...]
    y_ref[...] = y
    rid = i * RB + lax.broadcasted_iota(jnp.int32, (RB, 1), 0)
    msk = rid < ES_
    ym = jnp.where(msk, y, 0.0)
    y2 = jnp.where(msk, y * y, 0.0)
    acc_ref[...] += jnp.concatenate(
        [jnp.sum(ym, axis=0, keepdims=True),
         jnp.sum(y2, axis=0, keepdims=True)], axis=0)

    @pl.when(i == G1 - 1)
    def _():
        s_ref[...] = acc_ref[...]


def _mlp2(y1, sc1, sh1, w2, b2):
    return pl.pallas_call(
        _mlp2_body,
        grid=(G1,),
        in_specs=[pl.BlockSpec((RB, 512), lambda i: (i, 0)),
                  pl.BlockSpec((1, 512), lambda i: (0, 0)),
                  pl.BlockSpec((1, 512), lambda i: (0, 0)),
                  pl.BlockSpec((512, 256), lambda i: (0, 0)),
                  pl.BlockSpec((1, 256), lambda i: (0, 0))],
        out_specs=[pl.BlockSpec((RB, 256), lambda i: (i, 0)),
                   pl.BlockSpec((2, 256), lambda i: (0, 0))],
        out_shape=[jax.ShapeDtypeStruct((ESP, 256), jnp.float32),
                   jax.ShapeDtypeStruct((2, 256), jnp.float32)],
        scratch_shapes=[pltpu.VMEM((2, 256), jnp.float32)],
    )(y1, sc1, sh1, w2, b2)


def _mlp3_body(y2_ref, sc_ref, sh_ref, w3_ref, b3_ref, o_ref):
    x2 = jnp.maximum(y2_ref[...] * sc_ref[...] + sh_ref[...], 0.0)
    z = jnp.sum(x2 * w3_ref[...], axis=-1, keepdims=True) + b3_ref[...]
    o_ref[...] = jax.nn.sigmoid(z)


def _mlp3(y2, sc2, sh2, w3, b3):
    return pl.pallas_call(
        _mlp3_body,
        grid=(G1,),
        in_specs=[pl.BlockSpec((RB, 256), lambda i: (i, 0)),
                  pl.BlockSpec((1, 256), lambda i: (0, 0)),
                  pl.BlockSpec((1, 256), lambda i: (0, 0)),
                  pl.BlockSpec((1, 256), lambda i: (0, 0)),
                  pl.BlockSpec((1, 1), lambda i: (0, 0))],
        out_specs=pl.BlockSpec((RB, 1), lambda i: (i, 0)),
        out_shape=jax.ShapeDtypeStruct((ESP, 1), jnp.float32),
    )(y2, sc2, sh2, w3, b3)


def _prep(si, di):
    rp = jnp.searchsorted(
        di.astype(jnp.int32),
        jnp.arange(RPLEN, dtype=jnp.int32)).astype(jnp.int32)
    sip = jnp.concatenate([si.astype(jnp.int32),
                           jnp.zeros((528,), jnp.int32)])
    return sip, rp


def kernel(prot_h0, drug_h0, dis_h0, dt_src, dt_dst, dg_src, dg_dst,
           td_src, td_dst, gd_src, gd_dst, score_src, score_dst,
           Wk, Wq, Wv, Wa, lng, lnb, W1, bb1, g1, be1, W2, bb2, g2, be2,
           W3, bb3):
    dt = _prep(dt_src, dt_dst)
    dg = _prep(dg_src, dg_dst)
    td = _prep(td_src, td_dst)
    gd = _prep(gd_src, gd_dst)

    ph, dh, gh = prot_h0, drug_h0, dis_h0
    drug_feats, dis_feats = [], []
    for i in range(4):
        Wp = jnp.concatenate([Wq[i, 0], Wq[i, 1], Wk[i, 2], Wv[i, 2],
                              Wk[i, 3], Wv[i, 3]], axis=1)
        Wd = jnp.concatenate([Wk[i, 0], Wv[i, 0], Wq[i, 2]], axis=1)
        Wg = jnp.concatenate([Wk[i, 1], Wv[i, 1], Wq[i, 3]], axis=1)
        pj_p, pj_d, pj_g = _proj(ph, dh, gh, Wp, Wd, Wg)
        q_dt, q_dg = pj_p[:, 0:128], pj_p[:, 128:256]
        kv_td, kv_gd = pj_p[:, 256:512], pj_p[:, 512:768]
        kv_dt, q_td = pj_d[:, 0:256], pj_d[:, 256:384]
        kv_dg, q_gd = pj_g[:, 0:256], pj_g[:, 256:384]
        agg0 = _edge(kv_dt, q_dt, *dt)
        agg1 = _edge(kv_dg, q_dg, *dg)
        agg2 = _edge(kv_td, q_td, *td)
        agg3 = _edge(kv_gd, q_gd, *gd)
        ph, dh, gh = _post(agg0, agg1, agg2, agg3, ph, dh, gh,
                           Wa[i], lng[i][:, None, :], lnb[i][:, None, :])
        drug_feats.append(dh)
        dis_feats.append(gh)

    df = jnp.concatenate(drug_feats, axis=1)
    gf = jnp.concatenate(dis_feats, axis=1)
    ssrc = jnp.concatenate([score_src.astype(jnp.int32),
                            jnp.zeros((ESP - ES_,), jnp.int32)])
    sdst = jnp.concatenate([score_dst.astype(jnp.int32),
                            jnp.zeros((ESP - ES_,), jnp.int32)])
    hd, hg = _get_gath_call()(df, gf, ssrc, sdst)

    y1, s1 = _mlp1(hd, hg, W1[:512], W1[512:], bb1[None, :])
    mu1 = s1[0] / ES_
    var1 = s1[1] / ES_ - mu1 * mu1
    rs1 = 1.0 / jnp.sqrt(var1 + 1e-5)
    sc1 = (g1 * rs1)[None, :]
    sh1 = (be1 - g1 * rs1 * mu1)[None, :]

    y2, s2 = _mlp2(y1, sc1, sh1, W2, bb2[None, :])
    mu2 = s2[0] / ES_
    var2 = s2[1] / ES_ - mu2 * mu2
    rs2 = 1.0 / jnp.sqrt(var2 + 1e-5)
    sc2 = (g2 * rs2)[None, :]
    sh2 = (be2 - g2 * rs2 * mu2)[None, :]

    out = _mlp3(y2, sc2, sh2, W3[:, 0][None, :], bb3[None, :])
    return out.reshape(ESP)[:ES_]
